# Initial kernel scaffold; baseline (speedup 1.0000x reference)
import jax
import jax.numpy as jnp
from jax.experimental import pallas as pl


def kernel(x_nchw, weight_oihw, gamma, beta):
    raise NotImplementedError("write your pallas kernel here")


# speedup vs baseline: 3.6501x; 3.6501x over previous
import functools

import jax
import jax.numpy as jnp
from jax.experimental import pallas as pl
from jax.experimental.pallas import tpu as pltpu

EPS = 1e-5
NEG_SLOPE = 0.2


def _shift_lanes(x, s):
    L = x.shape[-1]
    k = s % L
    if k == 0:
        return x
    return jnp.concatenate([x[..., k:], x[..., :k]], axis=-1)


def _make_conv_stats_kernel(BA, Cin, Cout, W, HW):
    def body(x_ref, w_ref, y_ref, stats_ref):
        q = jax.lax.broadcasted_iota(jnp.int32, (1, HW), 1)
        wo = jax.lax.rem(q, W)
        vh = {-1: q >= W, 0: None, 1: q < HW - W}
        vw = {-1: wo > 0, 0: None, 1: wo < W - 1}

        w_mat = w_ref[...]
        sum_acc = None
        sq_acc = None
        for i in range(BA):
            xi = x_ref[i]
            taps = []
            for dh in (-1, 0, 1):
                for dw in (-1, 0, 1):
                    sh = _shift_lanes(xi, dh * W + dw)
                    m = vh[dh]
                    if vw[dw] is not None:
                        m = vw[dw] if m is None else (m & vw[dw])
                    if m is not None:
                        sh = jnp.where(m, sh, 0.0)
                    taps.append(sh)
            patches = jnp.concatenate(taps, axis=0)
            acc = jnp.dot(w_mat, patches,
                          preferred_element_type=jnp.float32)
            y_ref[i] = acc.astype(y_ref.dtype)
            if sum_acc is None:
                sum_acc = acc
                sq_acc = acc * acc
            else:
                sum_acc = sum_acc + acc
                sq_acc = sq_acc + acc * acc

        psum = jnp.sum(sum_acc, axis=1, keepdims=True)
        psq = jnp.sum(sq_acc, axis=1, keepdims=True)
        stats_ref[...] = jnp.concatenate([psum, psq], axis=1)

    return body


def _affine_lrelu_kernel(y_ref, scale_ref, shift_ref, o_ref):
    z = y_ref[...].astype(jnp.float32) * scale_ref[...] + shift_ref[...]
    o_ref[...] = jnp.where(z >= 0, z, NEG_SLOPE * z)


@functools.partial(jax.jit, static_argnames=("padding",))
def _conv_bn_lrelu(x_nchw, weight_oihw, gamma, beta, padding=1):
    N, Cin, H, W = x_nchw.shape
    Cout, _, KH, KW = weight_oihw.shape
    Ho = H + 2 * padding - KH + 1
    Wo = W + 2 * padding - KW + 1
    HW = Ho * Wo
    K = KH * KW * Cin

    BA = 8 if N % 8 == 0 else 1
    G = N // BA

    x_flat = x_nchw.reshape(N, Cin, H * W)
    w_mat = jnp.transpose(weight_oihw, (0, 2, 3, 1)).reshape(Cout, K)
    w_mat = w_mat.astype(jnp.float32)

    y, stats = pl.pallas_call(
        _make_conv_stats_kernel(BA, Cin, Cout, W, HW),
        out_shape=(jax.ShapeDtypeStruct((N, Cout, HW), jnp.bfloat16),
                   jax.ShapeDtypeStruct((G, Cout, 2), jnp.float32)),
        grid=(G,),
        in_specs=[
            pl.BlockSpec((BA, Cin, HW), lambda g: (g, 0, 0)),
            pl.BlockSpec((Cout, K), lambda g: (0, 0)),
        ],
        out_specs=(
            pl.BlockSpec((BA, Cout, HW), lambda g: (g, 0, 0)),
            pl.BlockSpec((None, Cout, 2), lambda g: (g, 0, 0)),
        ),
        compiler_params=pltpu.CompilerParams(
            dimension_semantics=("parallel",)),
    )(x_flat, w_mat)

    cnt = jnp.float32(N * HW)
    tot = jnp.sum(stats, axis=0)
    mean = tot[:, 0] / cnt
    var = tot[:, 1] / cnt - mean * mean
    scale = gamma.reshape(-1).astype(jnp.float32) * jax.lax.rsqrt(var + EPS)
    shift = beta.reshape(-1).astype(jnp.float32) - mean * scale
    scale = scale.reshape(Cout, 1)
    shift = shift.reshape(Cout, 1)

    BB = 16 if N % 16 == 0 else 1
    out_flat = pl.pallas_call(
        _affine_lrelu_kernel,
        out_shape=jax.ShapeDtypeStruct((N, Cout, HW), jnp.float32),
        grid=(N // BB,),
        in_specs=[
            pl.BlockSpec((BB, Cout, HW), lambda n: (n, 0, 0)),
            pl.BlockSpec((Cout, 1), lambda n: (0, 0)),
            pl.BlockSpec((Cout, 1), lambda n: (0, 0)),
        ],
        out_specs=pl.BlockSpec((BB, Cout, HW), lambda n: (n, 0, 0)),
        compiler_params=pltpu.CompilerParams(
            dimension_semantics=("parallel",)),
    )(y, scale, shift)

    return out_flat.reshape(N, Cout, Ho, Wo)


def kernel(x_nchw, weight_oihw, gamma, beta):
    return _conv_bn_lrelu(x_nchw, weight_oihw, gamma, beta, padding=1)
